# baseline (speedup 1.0000x reference)
import functools

import jax
import jax.numpy as jnp
from jax import lax
from jax.experimental import pallas as pl
from jax.experimental.pallas import tpu as pltpu

_N, _C, _H, _W = 8, 19, 512, 512
_HW = _H * _W
_LANES = 128
_ROWS = _HW // _LANES
_TR = 1024
_K = _ROWS // _TR
_CPAD = 24
_SUB = 8


def _tree_reduce(op, xs):
    while len(xs) > 1:
        xs = [op(xs[i], xs[i + 1]) for i in range(0, len(xs) - 1, 2)] + (
            [xs[-1]] if len(xs) % 2 else []
        )
    return xs[0]


def _main_body(x_ref, hist_ref, ssq_ref):
    k = pl.program_id(1)

    @pl.when(k == 0)
    def _init():
        hist_ref[...] = jnp.zeros_like(hist_ref)
        ssq_ref[...] = jnp.zeros_like(ssq_ref)

    zero = jnp.zeros((_SUB, _LANES), jnp.float32)

    def chunk(i, carry):
        h, q = carry
        base = pl.multiple_of(i * _SUB, _SUB)
        xs = [x_ref[0, c, pl.ds(base, _SUB), :] for c in range(_C)]
        m = _tree_reduce(jnp.maximum, xs)
        es = [jnp.exp(xs[c] - m) for c in range(_C)]
        s = _tree_reduce(lax.add, es)
        r2 = 1.0 / (s * s)
        h = tuple(
            h[c] + jnp.where(es[c] == 1.0, 1.0, 0.0) for c in range(_C)
        )
        q = tuple(q[c] + es[c] * es[c] * r2 for c in range(_C))
        return h, q

    h0 = tuple(zero for _ in range(_C))
    h, q = lax.fori_loop(0, _TR // _SUB, chunk, (h0, h0))

    zrow = jnp.zeros((1, _LANES), jnp.float32)
    hrows = [jnp.sum(h[c], axis=0, keepdims=True) for c in range(_C)]
    qrows = [jnp.sum(q[c], axis=0, keepdims=True) for c in range(_C)]
    zpad = [zrow] * (_CPAD - _C)
    hist_ref[...] = hist_ref[...] + jnp.concatenate(hrows + zpad, axis=0)[None]
    ssq_ref[...] = ssq_ref[...] + jnp.concatenate(qrows + zpad, axis=0)[None]


def _combine_body(h_ref, q_ref, o_ref):
    h = jnp.sum(h_ref[...], axis=2)
    q = jnp.sum(q_ref[...], axis=2)
    col = jax.lax.broadcasted_iota(jnp.int32, (_N, _CPAD), 1)
    mask = col < _C
    hadj = jnp.where(h == 0.0, 1.0, h)
    total = jnp.sum(jnp.where(mask, hadj, 0.0), axis=1, keepdims=True)
    w = jnp.exp(0.2 * (jnp.log(total) - jnp.log(hadj)))
    loss = -jnp.sum(jnp.where(mask, w * q, 0.0))
    o_ref[0, 0] = loss * (1.0 / (_N * _C * _H * _W))


def kernel(inputs):
    x = inputs.reshape(_N, _C, _ROWS, _LANES)
    hist, ssq = pl.pallas_call(
        _main_body,
        grid=(_N, _K),
        in_specs=[
            pl.BlockSpec((1, _C, _TR, _LANES), lambda n, k: (n, 0, k, 0)),
        ],
        out_specs=[
            pl.BlockSpec((1, _CPAD, _LANES), lambda n, k: (n, 0, 0)),
            pl.BlockSpec((1, _CPAD, _LANES), lambda n, k: (n, 0, 0)),
        ],
        out_shape=[
            jax.ShapeDtypeStruct((_N, _CPAD, _LANES), jnp.float32),
            jax.ShapeDtypeStruct((_N, _CPAD, _LANES), jnp.float32),
        ],
        compiler_params=pltpu.CompilerParams(
            dimension_semantics=("parallel", "arbitrary"),
        ),
    )(x)

    out = pl.pallas_call(
        _combine_body,
        out_shape=jax.ShapeDtypeStruct((1, 1), jnp.float32),
        out_specs=pl.BlockSpec(memory_space=pltpu.SMEM),
    )(hist, ssq)
    return out[0, 0]


# speedup vs baseline: 1.0919x; 1.0919x over previous
import functools

import jax
import jax.numpy as jnp
from jax import lax
from jax.experimental import pallas as pl
from jax.experimental.pallas import tpu as pltpu

_N, _C, _H, _W = 8, 19, 512, 512
_HW = _H * _W
_LANES = 128
_ROWS = _HW // _LANES
_TR = 1024
_K = _ROWS // _TR
_CPAD = 24
_SUB = 8


def _tree_reduce(op, xs):
    while len(xs) > 1:
        xs = [op(xs[i], xs[i + 1]) for i in range(0, len(xs) - 1, 2)] + (
            [xs[-1]] if len(xs) % 2 else []
        )
    return xs[0]


def _main_body(x_ref, hist_ref, ssq_ref):
    k = pl.program_id(1)

    @pl.when(k == 0)
    def _init():
        hist_ref[...] = jnp.zeros_like(hist_ref)
        ssq_ref[...] = jnp.zeros_like(ssq_ref)

    zero = jnp.zeros((_SUB, _LANES), jnp.float32)

    def chunk(i, carry):
        h, q = carry
        base = pl.multiple_of(i * (2 * _SUB), _SUB)
        hi = base + _SUB

        def lda(c):
            return x_ref[0, c, pl.ds(base, _SUB), :]

        def ldb(c):
            return x_ref[0, c, pl.ds(hi, _SUB), :]

        ma = lda(0)
        mb = ldb(0)
        for c in range(1, _C):
            ma = jnp.maximum(ma, lda(c))
            mb = jnp.maximum(mb, ldb(c))
        sa = jnp.zeros((_SUB, _LANES), jnp.float32)
        sb = jnp.zeros((_SUB, _LANES), jnp.float32)
        hn = []
        for c in range(_C):
            ea = jnp.exp(lda(c) - ma)
            eb = jnp.exp(ldb(c) - mb)
            sa = sa + ea
            sb = sb + eb
            hn.append(
                h[c]
                + (jnp.where(ea == 1.0, 1.0, 0.0) + jnp.where(eb == 1.0, 1.0, 0.0))
            )
        r2a = 1.0 / (sa * sa)
        r2b = 1.0 / (sb * sb)
        qn = []
        for c in range(_C):
            ea = jnp.exp(lda(c) - ma)
            eb = jnp.exp(ldb(c) - mb)
            qn.append(q[c] + (ea * ea * r2a + eb * eb * r2b))
        return tuple(hn), tuple(qn)

    h0 = tuple(zero for _ in range(_C))
    h, q = lax.fori_loop(0, _TR // (2 * _SUB), chunk, (h0, h0))

    zrow = jnp.zeros((1, _LANES), jnp.float32)
    hrows = [jnp.sum(h[c], axis=0, keepdims=True) for c in range(_C)]
    qrows = [jnp.sum(q[c], axis=0, keepdims=True) for c in range(_C)]
    zpad = [zrow] * (_CPAD - _C)
    hist_ref[...] = hist_ref[...] + jnp.concatenate(hrows + zpad, axis=0)[None]
    ssq_ref[...] = ssq_ref[...] + jnp.concatenate(qrows + zpad, axis=0)[None]


def _combine_body(h_ref, q_ref, o_ref):
    h = jnp.sum(h_ref[...], axis=2)
    q = jnp.sum(q_ref[...], axis=2)
    col = jax.lax.broadcasted_iota(jnp.int32, (_N, _CPAD), 1)
    mask = col < _C
    hadj = jnp.where(h == 0.0, 1.0, h)
    total = jnp.sum(jnp.where(mask, hadj, 0.0), axis=1, keepdims=True)
    w = jnp.exp(0.2 * (jnp.log(total) - jnp.log(hadj)))
    loss = -jnp.sum(jnp.where(mask, w * q, 0.0))
    o_ref[0, 0] = loss * (1.0 / (_N * _C * _H * _W))


def kernel(inputs):
    x = inputs.reshape(_N, _C, _ROWS, _LANES)
    hist, ssq = pl.pallas_call(
        _main_body,
        grid=(_N, _K),
        in_specs=[
            pl.BlockSpec((1, _C, _TR, _LANES), lambda n, k: (n, 0, k, 0)),
        ],
        out_specs=[
            pl.BlockSpec((1, _CPAD, _LANES), lambda n, k: (n, 0, 0)),
            pl.BlockSpec((1, _CPAD, _LANES), lambda n, k: (n, 0, 0)),
        ],
        out_shape=[
            jax.ShapeDtypeStruct((_N, _CPAD, _LANES), jnp.float32),
            jax.ShapeDtypeStruct((_N, _CPAD, _LANES), jnp.float32),
        ],
        compiler_params=pltpu.CompilerParams(
            dimension_semantics=("parallel", "arbitrary"),
        ),
    )(x)

    out = pl.pallas_call(
        _combine_body,
        out_shape=jax.ShapeDtypeStruct((1, 1), jnp.float32),
        out_specs=pl.BlockSpec(memory_space=pltpu.SMEM),
    )(hist, ssq)
    return out[0, 0]


# speedup vs baseline: 1.1161x; 1.0222x over previous
import jax
import jax.numpy as jnp
from jax.experimental import pallas as pl
from jax.experimental.pallas import tpu as pltpu

_N, _C, _H, _W = 8, 19, 512, 512
_HW = _H * _W
_LANES = 128
_ROWS = _HW // _LANES
_TR = 1024
_K = _ROWS // _TR
_CPAD = 24


def _main_body(x_ref, o_ref, e_ref, hacc_ref, qacc_ref):
    n = pl.program_id(0)
    k = pl.program_id(1)

    @pl.when((n == 0) & (k == 0))
    def _init_out():
        o_ref[0, 0] = 0.0

    @pl.when(k == 0)
    def _init_acc():
        hacc_ref[...] = jnp.zeros_like(hacc_ref)
        qacc_ref[...] = jnp.zeros_like(qacc_ref)

    x = x_ref[0]

    m = x[0]
    for c in range(1, _C):
        m = jnp.maximum(m, x[c])

    s = jnp.zeros_like(m)
    hrows = []
    for c in range(_C):
        e = jnp.exp(x[c] - m)
        e_ref[c] = e
        s = s + e
        hrows.append(jnp.sum(jnp.where(e == 1.0, 1.0, 0.0), axis=0, keepdims=True))
    r2 = 1.0 / (s * s)

    qrows = []
    for c in range(_C):
        e = e_ref[c]
        qrows.append(jnp.sum(e * e * r2, axis=0, keepdims=True))

    zpad = [jnp.zeros((1, _LANES), jnp.float32)] * (_CPAD - _C)
    hacc_ref[...] = hacc_ref[...] + jnp.concatenate(hrows + zpad, axis=0)
    qacc_ref[...] = qacc_ref[...] + jnp.concatenate(qrows + zpad, axis=0)

    @pl.when(k == _K - 1)
    def _epilogue():
        h = jnp.sum(hacc_ref[...], axis=1, keepdims=True)
        q = jnp.sum(qacc_ref[...], axis=1, keepdims=True)
        row = jax.lax.broadcasted_iota(jnp.int32, (_CPAD, 1), 0)
        mask = row < _C
        hadj = jnp.where(h == 0.0, 1.0, h)
        total = jnp.sum(jnp.where(mask, hadj, 0.0), axis=0, keepdims=True)
        w = jnp.exp(0.2 * (jnp.log(total) - jnp.log(hadj)))
        contrib = -jnp.sum(jnp.where(mask, w * q, 0.0))
        o_ref[0, 0] += contrib * (1.0 / (_N * _C * _H * _W))


def kernel(inputs):
    x = inputs.reshape(_N, _C, _ROWS, _LANES)
    out = pl.pallas_call(
        _main_body,
        grid=(_N, _K),
        in_specs=[
            pl.BlockSpec((1, _C, _TR, _LANES), lambda n, k: (n, 0, k, 0)),
        ],
        out_specs=pl.BlockSpec(memory_space=pltpu.SMEM),
        out_shape=jax.ShapeDtypeStruct((1, 1), jnp.float32),
        scratch_shapes=[
            pltpu.VMEM((_C, _TR, _LANES), jnp.float32),
            pltpu.VMEM((_CPAD, _LANES), jnp.float32),
            pltpu.VMEM((_CPAD, _LANES), jnp.float32),
        ],
        compiler_params=pltpu.CompilerParams(
            dimension_semantics=("arbitrary", "arbitrary"),
        ),
    )(x)
    return out[0, 0]
